# baseline (speedup 1.0000x reference)
import functools

import jax
import jax.numpy as jnp
from jax.experimental import pallas as pl
from jax.experimental.pallas import tpu as pltpu

DIM = 4096
N_MODULES = 64
N_ACTIVE = 8
N_OUT = N_MODULES * N_ACTIVE


def _router_kernel(x_ref, wt_ref, b_ref, sel_ref, logits_ref):
    t = x_ref.shape[0]
    acc = jax.lax.dot_general(
        x_ref[...], wt_ref[...],
        (((1,), (0,)), ((), ())),
        preferred_element_type=jnp.float32,
    )
    logits = acc + b_ref[...]
    lg = logits.reshape(t, N_ACTIVE, N_MODULES)
    logits_ref[...] = lg
    mx = jnp.max(lg, axis=-1, keepdims=True)
    iota = jax.lax.broadcasted_iota(jnp.int32, lg.shape, 2)
    sel = jnp.min(jnp.where(lg == mx, iota, N_MODULES), axis=-1)
    sel_ref[...] = sel.astype(jnp.int32)


@jax.jit
def kernel(x, W, b):
    n_tokens = x.shape[0]
    block_t = 256
    grid = (n_tokens // block_t,)
    wt = W.T
    b2 = b.reshape(1, N_OUT)
    sel, logits = pl.pallas_call(
        _router_kernel,
        grid=grid,
        in_specs=[
            pl.BlockSpec((block_t, DIM), lambda i: (i, 0)),
            pl.BlockSpec((DIM, N_OUT), lambda i: (0, 0)),
            pl.BlockSpec((1, N_OUT), lambda i: (0, 0)),
        ],
        out_specs=[
            pl.BlockSpec((block_t, N_ACTIVE), lambda i: (i, 0)),
            pl.BlockSpec((block_t, N_ACTIVE, N_MODULES), lambda i: (i, 0, 0)),
        ],
        out_shape=[
            jax.ShapeDtypeStruct((n_tokens, N_ACTIVE), jnp.int32),
            jax.ShapeDtypeStruct((n_tokens, N_ACTIVE, N_MODULES), jnp.float32),
        ],
    )(x, wt, b2)
    return (sel, logits)


# speedup vs baseline: 1.2019x; 1.2019x over previous
import jax
import jax.numpy as jnp
from jax.experimental import pallas as pl
from jax.experimental.pallas import tpu as pltpu

DIM = 4096
N_MODULES = 64
N_ACTIVE = 8
N_OUT = N_MODULES * N_ACTIVE


def _router_kernel(x_ref, wt_ref, b_ref, sel_ref, logits_ref):
    t = x_ref.shape[0]
    acc = jax.lax.dot_general(
        x_ref[...], wt_ref[...],
        (((1,), (0,)), ((), ())),
        preferred_element_type=jnp.float32,
    )
    logits = acc + b_ref[...]
    logits_ref[...] = logits
    sel = jnp.zeros((t, N_ACTIVE), jnp.float32)
    col = jax.lax.broadcasted_iota(jnp.int32, (t, N_ACTIVE), 1)
    iota = jax.lax.broadcasted_iota(
        jnp.int32, (t, N_MODULES), 1).astype(jnp.float32)
    for a in range(N_ACTIVE):
        g = logits[:, a * N_MODULES:(a + 1) * N_MODULES]
        mx = jnp.max(g, axis=1, keepdims=True)
        ga = jnp.min(jnp.where(g == mx, iota, float(N_MODULES)),
                     axis=1, keepdims=True)
        sel = jnp.where(col == a, ga, sel)
    sel_ref[...] = sel.astype(jnp.int32)


@jax.jit
def kernel(x, W, b):
    n_tokens = x.shape[0]
    block_t = 256
    grid = (n_tokens // block_t,)
    wt = W.T
    b2 = b.reshape(1, N_OUT)
    sel, logits = pl.pallas_call(
        _router_kernel,
        grid=grid,
        in_specs=[
            pl.BlockSpec((block_t, DIM), lambda i: (i, 0)),
            pl.BlockSpec((DIM, N_OUT), lambda i: (0, 0)),
            pl.BlockSpec((1, N_OUT), lambda i: (0, 0)),
        ],
        out_specs=[
            pl.BlockSpec((block_t, N_ACTIVE), lambda i: (i, 0)),
            pl.BlockSpec((block_t, N_OUT), lambda i: (i, 0)),
        ],
        out_shape=[
            jax.ShapeDtypeStruct((n_tokens, N_ACTIVE), jnp.int32),
            jax.ShapeDtypeStruct((n_tokens, N_OUT), jnp.float32),
        ],
    )(x, wt, b2)
    return (sel, logits.reshape(n_tokens, N_ACTIVE, N_MODULES))


# speedup vs baseline: 1.3316x; 1.1079x over previous
import jax
import jax.numpy as jnp
from jax.experimental import pallas as pl
from jax.experimental.pallas import tpu as pltpu

DIM = 4096
N_MODULES = 64
N_ACTIVE = 8
N_OUT = N_MODULES * N_ACTIVE


def _router_kernel(x_ref, wt_ref, b_ref, sel_ref, logits_ref):
    t = x_ref.shape[0]
    acc = jax.lax.dot_general(
        x_ref[...], wt_ref[...],
        (((1,), (0,)), ((), ())),
        preferred_element_type=jnp.float32,
    )
    logits = acc + b_ref[...]
    logits_ref[...] = logits
    sel = jnp.zeros((t, N_ACTIVE), jnp.float32)
    col = jax.lax.broadcasted_iota(jnp.int32, (t, N_ACTIVE), 1)
    iota = jax.lax.broadcasted_iota(
        jnp.int32, (t, N_MODULES), 1).astype(jnp.float32)
    for a in range(N_ACTIVE):
        g = logits[:, a * N_MODULES:(a + 1) * N_MODULES]
        mx = jnp.max(g, axis=1, keepdims=True)
        ga = jnp.min(jnp.where(g == mx, iota, float(N_MODULES)),
                     axis=1, keepdims=True)
        sel = jnp.where(col == a, ga, sel)
    sel_ref[...] = sel.astype(jnp.int32)


@jax.jit
def kernel(x, W, b):
    n_tokens = x.shape[0]
    block_t = 512
    grid = (n_tokens // block_t,)
    wt = W.T
    b2 = b.reshape(1, N_OUT)
    sel, logits = pl.pallas_call(
        _router_kernel,
        grid=grid,
        in_specs=[
            pl.BlockSpec((block_t, DIM), lambda i: (i, 0)),
            pl.BlockSpec((DIM, N_OUT), lambda i: (0, 0)),
            pl.BlockSpec((1, N_OUT), lambda i: (0, 0)),
        ],
        out_specs=[
            pl.BlockSpec((block_t, N_ACTIVE), lambda i: (i, 0)),
            pl.BlockSpec((block_t, N_OUT), lambda i: (i, 0)),
        ],
        out_shape=[
            jax.ShapeDtypeStruct((n_tokens, N_ACTIVE), jnp.int32),
            jax.ShapeDtypeStruct((n_tokens, N_OUT), jnp.float32),
        ],
    )(x, wt, b2)
    return (sel, logits.reshape(n_tokens, N_ACTIVE, N_MODULES))


# speedup vs baseline: 1.3442x; 1.0094x over previous
import jax
import jax.numpy as jnp
from jax.experimental import pallas as pl
from jax.experimental.pallas import tpu as pltpu

DIM = 4096
N_MODULES = 64
N_ACTIVE = 8
N_OUT = N_MODULES * N_ACTIVE


def _router_kernel(x_ref, wt_ref, b_ref, sel_ref, logits_ref):
    t = x_ref.shape[0]
    acc = jax.lax.dot_general(
        x_ref[...], wt_ref[...],
        (((1,), (0,)), ((), ())),
        preferred_element_type=jnp.float32,
    )
    logits = acc + b_ref[...]
    logits_ref[...] = logits
    sel = jnp.zeros((t, N_ACTIVE), jnp.float32)
    col = jax.lax.broadcasted_iota(jnp.int32, (t, N_ACTIVE), 1)
    iota = jax.lax.broadcasted_iota(
        jnp.int32, (t, N_MODULES), 1).astype(jnp.float32)
    for a in range(N_ACTIVE):
        g = logits[:, a * N_MODULES:(a + 1) * N_MODULES]
        mx = jnp.max(g, axis=1, keepdims=True)
        ga = jnp.min(jnp.where(g == mx, iota, float(N_MODULES)),
                     axis=1, keepdims=True)
        sel = jnp.where(col == a, ga, sel)
    sel_ref[...] = sel.astype(jnp.int32)


@jax.jit
def kernel(x, W, b):
    n_tokens = x.shape[0]
    block_t = 512
    grid = (n_tokens // block_t,)
    wt = W.T
    b2 = b.reshape(1, N_OUT)
    sel, logits = pl.pallas_call(
        _router_kernel,
        grid=grid,
        compiler_params=pltpu.CompilerParams(
            dimension_semantics=("parallel",),
        ),
        in_specs=[
            pl.BlockSpec((block_t, DIM), lambda i: (i, 0)),
            pl.BlockSpec((DIM, N_OUT), lambda i: (0, 0)),
            pl.BlockSpec((1, N_OUT), lambda i: (0, 0)),
        ],
        out_specs=[
            pl.BlockSpec((block_t, N_ACTIVE), lambda i: (i, 0)),
            pl.BlockSpec((block_t, N_OUT), lambda i: (i, 0)),
        ],
        out_shape=[
            jax.ShapeDtypeStruct((n_tokens, N_ACTIVE), jnp.int32),
            jax.ShapeDtypeStruct((n_tokens, N_OUT), jnp.float32),
        ],
    )(x, wt, b2)
    return (sel, logits.reshape(n_tokens, N_ACTIVE, N_MODULES))
